# baseline (speedup 1.0000x reference)
import functools

import jax
import jax.numpy as jnp
from jax import lax
from jax.experimental import pallas as pl
from jax.experimental.pallas import tpu as pltpu
from jax.experimental.pallas import tpu_sc as plsc

_B = 16384
_D = 16
_CHUNK = 128
_BM = 8192


def _make_sc_gather():
  info = plsc.get_sparse_core_info()
  nw = info.num_cores * info.num_subcores
  b_per_w = _B // nw
  n_chunks = b_per_w // _CHUNK
  halves = _BM // b_per_w
  mesh = plsc.VectorSubcoreMesh(core_axis_name="c", subcore_axis_name="s")

  @functools.partial(
      pl.kernel,
      mesh=mesh,
      out_type=[
          jax.ShapeDtypeStruct((_B // 8, 8 * _D), jnp.float32),
          jax.ShapeDtypeStruct((_B // 8, 8 * _D), jnp.float32),
      ],
      scratch_types=[
          pltpu.VMEM((b_per_w,), jnp.int32),
          pltpu.VMEM((b_per_w,), jnp.int32),
          pltpu.VMEM((b_per_w, _D), jnp.float32),
          pltpu.VMEM((b_per_w, _D), jnp.float32),
          pltpu.SemaphoreType.DMA,
          [pltpu.SemaphoreType.DMA] * (_B // (32 * _CHUNK)),
      ],
      compiler_params=pltpu.CompilerParams(use_tc_tiling_on_sc=False),
  )
  def sc_gather(ctab, gtab, cidx, gidx, cout, gout,
                cidx_v, gidx_v, crow_v, grow_v, out_sem, chunk_sems):
    wid = lax.axis_index("s") * info.num_cores + lax.axis_index("c")
    base = wid * b_per_w
    idx_in = [
        pltpu.async_copy(cidx.at[pl.ds(base, b_per_w)], cidx_v, out_sem),
        pltpu.async_copy(gidx.at[pl.ds(base, b_per_w)], gidx_v, out_sem),
    ]
    for cp in idx_in:
      cp.wait()
    gathers = []
    for j in range(n_chunks):
      sl = pl.ds(j * _CHUNK, _CHUNK)
      gathers.append(pltpu.async_copy(
          ctab.at[cidx_v.at[sl]], crow_v.at[sl], chunk_sems[j]))
      gathers.append(pltpu.async_copy(
          gtab.at[gidx_v.at[sl]], grow_v.at[sl], chunk_sems[j]))
    slab = _BM // 8
    outs = []
    for j in range(n_chunks):
      sl = pl.ds(j * _CHUNK, _CHUNK)
      t = wid * n_chunks + j
      r = (t * _CHUNK) % _BM
      row0 = ((t * _CHUNK) // _BM) * slab + r % slab
      lane0 = (r // slab) * _D
      gathers[2 * j].wait()
      gathers[2 * j + 1].wait()
      outs.append(pltpu.async_copy(
          crow_v.at[sl],
          cout.at[pl.ds(row0, _CHUNK), pl.ds(lane0, _D)], out_sem))
      outs.append(pltpu.async_copy(
          grow_v.at[sl],
          gout.at[pl.ds(row0, _CHUNK), pl.ds(lane0, _D)], out_sem))
    for cp in outs:
      cp.wait()

  return sc_gather


_SC_GATHER_CACHE = []


def _sc_gather(*args):
  if not _SC_GATHER_CACHE:
    _SC_GATHER_CACHE.append(_make_sc_gather())
  return _SC_GATHER_CACHE[0](*args)


def _mlp_body(wt_ref, cr_ref, c_ref, g_ref, w1_ref, b1_ref, w2_ref, b2_ref,
              out_ref):
  w1cg = w1_ref[2:2 + 2 * _D, :]
  w1wt = w1_ref[0:1, :]
  w1cr = w1_ref[1:2, :]
  b1 = jnp.reshape(b1_ref[...], (1, 256))
  w2 = w2_ref[...]
  b2 = jnp.reshape(b2_ref[...], (1, 128))
  c8 = c_ref[...]
  g8 = g_ref[...]
  cg = jnp.concatenate(
      [jnp.concatenate([c8[:, k * _D:(k + 1) * _D],
                        g8[:, k * _D:(k + 1) * _D]], axis=1)
       for k in range(8)], axis=0)
  pre = jnp.dot(cg.astype(jnp.bfloat16), w1cg.astype(jnp.bfloat16),
                preferred_element_type=jnp.float32)
  pre += jnp.reshape(wt_ref[...], (_BM, 1)) * w1wt
  pre += jnp.reshape(cr_ref[...], (_BM, 1)) * w1cr
  pre += b1
  h = jnp.maximum(pre, 0.0)
  emb = jnp.dot(h.astype(jnp.bfloat16), w2.astype(jnp.bfloat16),
                preferred_element_type=jnp.float32)
  emb += b2
  n2 = jnp.sum(emb * emb, axis=1, keepdims=True)
  out_ref[...] = emb * jax.lax.rsqrt(jnp.maximum(n2, 1e-24))


def _mlp(wt, cr, c, g, w1, b1, w2, b2):
  grid = (_B // _BM,)
  return pl.pallas_call(
      _mlp_body,
      grid=grid,
      in_specs=[
          pl.BlockSpec((_BM,), lambda i: (i,)),
          pl.BlockSpec((_BM,), lambda i: (i,)),
          pl.BlockSpec((_BM // 8, 8 * _D), lambda i: (i, 0)),
          pl.BlockSpec((_BM // 8, 8 * _D), lambda i: (i, 0)),
          pl.BlockSpec((2 + 2 * _D, 256), lambda i: (0, 0)),
          pl.BlockSpec((256,), lambda i: (0,)),
          pl.BlockSpec((256, 128), lambda i: (0, 0)),
          pl.BlockSpec((128,), lambda i: (0,)),
      ],
      out_specs=pl.BlockSpec((_BM, 128), lambda i: (i, 0)),
      out_shape=jax.ShapeDtypeStruct((_B, 128), jnp.float32),
      compiler_params=pltpu.CompilerParams(
          dimension_semantics=("parallel",)),
  )(wt, cr, c, g, w1, b1, w2, b2)


def kernel(watch_time, completion_rate, country_idx, fav_genre_idx,
           country_table, genre_table, W1, b1, W2, b2):
  cidx = country_idx.astype(jnp.int32)
  gidx = fav_genre_idx.astype(jnp.int32)
  c8, g8 = _sc_gather(country_table, genre_table, cidx, gidx)
  return _mlp(watch_time, completion_rate, c8, g8, W1, b1, W2, b2)


# speedup vs baseline: 1.0132x; 1.0132x over previous
import functools

import jax
import jax.numpy as jnp
from jax import lax
from jax.experimental import pallas as pl
from jax.experimental.pallas import tpu as pltpu
from jax.experimental.pallas import tpu_sc as plsc

_B = 16384
_D = 16
_CHUNK = 128
_BM = 4096


def _make_sc_gather():
  info = plsc.get_sparse_core_info()
  nw = info.num_cores * info.num_subcores
  b_per_w = _B // nw
  n_chunks = b_per_w // _CHUNK
  halves = _BM // b_per_w
  mesh = plsc.VectorSubcoreMesh(core_axis_name="c", subcore_axis_name="s")

  @functools.partial(
      pl.kernel,
      mesh=mesh,
      out_type=[
          jax.ShapeDtypeStruct((_B // 8, 8 * _D), jnp.float32),
          jax.ShapeDtypeStruct((_B // 8, 8 * _D), jnp.float32),
      ],
      scratch_types=[
          pltpu.VMEM((b_per_w,), jnp.int32),
          pltpu.VMEM((b_per_w,), jnp.int32),
          pltpu.VMEM((b_per_w, _D), jnp.float32),
          pltpu.VMEM((b_per_w, _D), jnp.float32),
          pltpu.SemaphoreType.DMA,
          [pltpu.SemaphoreType.DMA] * (_B // (32 * _CHUNK)),
      ],
      compiler_params=pltpu.CompilerParams(use_tc_tiling_on_sc=False),
  )
  def sc_gather(ctab, gtab, cidx, gidx, cout, gout,
                cidx_v, gidx_v, crow_v, grow_v, out_sem, chunk_sems):
    wid = lax.axis_index("s") * info.num_cores + lax.axis_index("c")
    base = wid * b_per_w
    idx_in = [
        pltpu.async_copy(cidx.at[pl.ds(base, b_per_w)], cidx_v, out_sem),
        pltpu.async_copy(gidx.at[pl.ds(base, b_per_w)], gidx_v, out_sem),
    ]
    for cp in idx_in:
      cp.wait()
    gathers = []
    for j in range(n_chunks):
      sl = pl.ds(j * _CHUNK, _CHUNK)
      gathers.append(pltpu.async_copy(
          ctab.at[cidx_v.at[sl]], crow_v.at[sl], chunk_sems[j]))
      gathers.append(pltpu.async_copy(
          gtab.at[gidx_v.at[sl]], grow_v.at[sl], chunk_sems[j]))
    slab = _BM // 8
    outs = []
    for j in range(n_chunks):
      sl = pl.ds(j * _CHUNK, _CHUNK)
      t = wid * n_chunks + j
      r = (t * _CHUNK) % _BM
      row0 = ((t * _CHUNK) // _BM) * slab + r % slab
      lane0 = (r // slab) * _D
      gathers[2 * j].wait()
      gathers[2 * j + 1].wait()
      outs.append(pltpu.async_copy(
          crow_v.at[sl],
          cout.at[pl.ds(row0, _CHUNK), pl.ds(lane0, _D)], out_sem))
      outs.append(pltpu.async_copy(
          grow_v.at[sl],
          gout.at[pl.ds(row0, _CHUNK), pl.ds(lane0, _D)], out_sem))
    for cp in outs:
      cp.wait()

  return sc_gather


_SC_GATHER_CACHE = []


def _sc_gather(*args):
  if not _SC_GATHER_CACHE:
    _SC_GATHER_CACHE.append(_make_sc_gather())
  return _SC_GATHER_CACHE[0](*args)


def _mlp_body(wt_ref, cr_ref, c_ref, g_ref, w1_ref, b1_ref, w2_ref, b2_ref,
              out_ref):
  w1cg = w1_ref[2:2 + 2 * _D, :]
  w1wt = w1_ref[0:1, :]
  w1cr = w1_ref[1:2, :]
  b1 = jnp.reshape(b1_ref[...], (1, 256))
  w2 = w2_ref[...]
  b2 = jnp.reshape(b2_ref[...], (1, 128))
  c8 = c_ref[...]
  g8 = g_ref[...]
  cg = jnp.concatenate(
      [jnp.concatenate([c8[:, k * _D:(k + 1) * _D],
                        g8[:, k * _D:(k + 1) * _D]], axis=1)
       for k in range(8)], axis=0)
  pre = jnp.dot(cg.astype(jnp.bfloat16), w1cg.astype(jnp.bfloat16),
                preferred_element_type=jnp.float32)
  pre += jnp.reshape(wt_ref[...], (_BM, 1)) * w1wt
  pre += jnp.reshape(cr_ref[...], (_BM, 1)) * w1cr
  pre += b1
  h = jnp.maximum(pre, 0.0)
  emb = jnp.dot(h.astype(jnp.bfloat16), w2.astype(jnp.bfloat16),
                preferred_element_type=jnp.float32)
  emb += b2
  n2 = jnp.sum(emb * emb, axis=1, keepdims=True)
  out_ref[...] = emb * jax.lax.rsqrt(jnp.maximum(n2, 1e-24))


def _mlp(wt, cr, c, g, w1, b1, w2, b2):
  grid = (_B // _BM,)
  return pl.pallas_call(
      _mlp_body,
      grid=grid,
      in_specs=[
          pl.BlockSpec((_BM,), lambda i: (i,)),
          pl.BlockSpec((_BM,), lambda i: (i,)),
          pl.BlockSpec((_BM // 8, 8 * _D), lambda i: (i, 0)),
          pl.BlockSpec((_BM // 8, 8 * _D), lambda i: (i, 0)),
          pl.BlockSpec((2 + 2 * _D, 256), lambda i: (0, 0)),
          pl.BlockSpec((256,), lambda i: (0,)),
          pl.BlockSpec((256, 128), lambda i: (0, 0)),
          pl.BlockSpec((128,), lambda i: (0,)),
      ],
      out_specs=pl.BlockSpec((_BM, 128), lambda i: (i, 0)),
      out_shape=jax.ShapeDtypeStruct((_B, 128), jnp.float32),
      compiler_params=pltpu.CompilerParams(
          dimension_semantics=("parallel",)),
  )(wt, cr, c, g, w1, b1, w2, b2)


def kernel(watch_time, completion_rate, country_idx, fav_genre_idx,
           country_table, genre_table, W1, b1, W2, b2):
  cidx = country_idx.astype(jnp.int32)
  gidx = fav_genre_idx.astype(jnp.int32)
  c8, g8 = _sc_gather(country_table, genre_table, cidx, gidx)
  return _mlp(watch_time, completion_rate, c8, g8, W1, b1, W2, b2)
